# baseline (speedup 1.0000x reference)
import jax
import jax.numpy as jnp
from jax.experimental import pallas as pl
from jax.experimental.pallas import tpu as pltpu

_B, _N, _M, _D = 4, 4096, 4096, 3
_TN = 2048
_NB = _N // _TN
_K = 14


def _chamfer_block_kernel(u_ref, xn_ref, yt_ref, yn_ref, o1_ref, o2_ref, m2_ref):
    i = pl.program_id(1)

    u = u_ref[0]
    xn = xn_ref[0]
    yt = yt_ref[0]
    yn = yn_ref[0]

    t = jax.lax.dot_general(
        u, yt, (((1,), (0,)), ((), ())), preferred_element_type=jnp.float32
    )

    r1 = jnp.min(t, axis=1)[:, None] + xn
    s1 = jnp.sum(r1, keepdims=True)[:1, :1]

    d2 = t + xn
    m2 = jnp.min(d2, axis=0, keepdims=True)

    @pl.when(i == 0)
    def _init():
        o1_ref[...] = s1[None]
        m2_ref[...] = m2

    @pl.when(i > 0)
    def _acc():
        o1_ref[...] += s1[None]
        m2_ref[...] = jnp.minimum(m2_ref[...], m2)

    @pl.when(i == _NB - 1)
    def _flush_m2():
        o2_ref[...] = jnp.sum(m2_ref[...], keepdims=True)[None]


def _split_hi_lo(a):
    bits = jax.lax.bitcast_convert_type(a, jnp.uint32)
    hi = jax.lax.bitcast_convert_type(
        bits & jnp.uint32(0xFFFF0000), jnp.float32
    )
    lo = a - hi
    return hi.astype(jnp.bfloat16), lo.astype(jnp.bfloat16)


@jax.jit
def kernel(input1, input2):
    u = -2.0 * input1
    uh, ul = _split_hi_lo(u)
    onesx = jnp.ones((_B, _N, 1), jnp.bfloat16)
    ua = jnp.concatenate([uh, uh, ul, ul, onesx, onesx], axis=2)
    xn = jnp.sum(input1 * input1, axis=2, keepdims=True)
    yt = jnp.transpose(input2, (0, 2, 1))
    yh, yl = _split_hi_lo(yt)
    yn = jnp.sum(input2 * input2, axis=2)[:, None, :]
    ynh, ynl = _split_hi_lo(yn)
    ya = jnp.concatenate([yh, yl, yh, yl, ynh, ynl], axis=1)
    s1, s2 = pl.pallas_call(
        _chamfer_block_kernel,
        grid=(_B, _NB),
        in_specs=[
            pl.BlockSpec((1, _TN, _K), lambda b, i: (b, i, 0)),
            pl.BlockSpec((1, _TN, 1), lambda b, i: (b, i, 0)),
            pl.BlockSpec((1, _K, _M), lambda b, i: (b, 0, 0)),
            pl.BlockSpec((1, 1, _M), lambda b, i: (b, 0, 0)),
        ],
        out_specs=[
            pl.BlockSpec((1, 1, 1), lambda b, i: (b, 0, 0)),
            pl.BlockSpec((1, 1, 1), lambda b, i: (b, 0, 0)),
        ],
        out_shape=[
            jax.ShapeDtypeStruct((_B, 1, 1), jnp.float32),
            jax.ShapeDtypeStruct((_B, 1, 1), jnp.float32),
        ],
        scratch_shapes=[pltpu.VMEM((1, _M), jnp.float32)],
    )(ua, xn, ya, yn)
    return jnp.sum(s1) * (1.0 / (_B * _N)) + jnp.sum(s2) * (1.0 / (_B * _M))


# speedup vs baseline: 1.1800x; 1.1800x over previous
import jax
import jax.numpy as jnp
from jax.experimental import pallas as pl
from jax.experimental.pallas import tpu as pltpu

_B, _N, _M, _D = 4, 4096, 4096, 3
_TN = 2048
_NB = _N // _TN
_K = 12


def _chamfer_block_kernel(u_ref, xn_ref, yt_ref, yn_ref, o1_ref, o2_ref, m2_ref):
    i = pl.program_id(1)

    u = u_ref[0]
    xn = xn_ref[0]
    yt = yt_ref[0]
    yn = yn_ref[0]

    g = jax.lax.dot_general(
        u, yt, (((1,), (0,)), ((), ())), preferred_element_type=jnp.float32
    )

    d = (g + xn) + yn

    r1 = jnp.min(d, axis=1)[:, None]
    s1 = jnp.sum(r1, keepdims=True)[:1, :1]

    m2 = jnp.min(d, axis=0, keepdims=True)

    @pl.when(i == 0)
    def _init():
        o1_ref[...] = s1[None]
        m2_ref[...] = m2

    @pl.when(i > 0)
    def _acc():
        o1_ref[...] += s1[None]
        m2_ref[...] = jnp.minimum(m2_ref[...], m2)

    @pl.when(i == _NB - 1)
    def _flush_m2():
        o2_ref[...] = jnp.sum(m2_ref[...], keepdims=True)[None]


def _split_hi_lo(a):
    bits = jax.lax.bitcast_convert_type(a, jnp.uint32)
    hi = jax.lax.bitcast_convert_type(
        bits & jnp.uint32(0xFFFF0000), jnp.float32
    )
    lo = a - hi
    return hi.astype(jnp.bfloat16), lo.astype(jnp.bfloat16)


@jax.jit
def kernel(input1, input2):
    u = -2.0 * input1
    uh, ul = _split_hi_lo(u)
    ua = jnp.concatenate([uh, uh, ul, ul], axis=2)
    xn = jnp.sum(input1 * input1, axis=2, keepdims=True)
    yt = jnp.transpose(input2, (0, 2, 1))
    yh, yl = _split_hi_lo(yt)
    ya = jnp.concatenate([yh, yl, yh, yl], axis=1)
    yn = jnp.sum(input2 * input2, axis=2)[:, None, :]
    s1, s2 = pl.pallas_call(
        _chamfer_block_kernel,
        grid=(_B, _NB),
        in_specs=[
            pl.BlockSpec((1, _TN, _K), lambda b, i: (b, i, 0)),
            pl.BlockSpec((1, _TN, 1), lambda b, i: (b, i, 0)),
            pl.BlockSpec((1, _K, _M), lambda b, i: (b, 0, 0)),
            pl.BlockSpec((1, 1, _M), lambda b, i: (b, 0, 0)),
        ],
        out_specs=[
            pl.BlockSpec((1, 1, 1), lambda b, i: (b, 0, 0)),
            pl.BlockSpec((1, 1, 1), lambda b, i: (b, 0, 0)),
        ],
        out_shape=[
            jax.ShapeDtypeStruct((_B, 1, 1), jnp.float32),
            jax.ShapeDtypeStruct((_B, 1, 1), jnp.float32),
        ],
        scratch_shapes=[pltpu.VMEM((1, _M), jnp.float32)],
    )(ua, xn, ya, yn)
    return jnp.sum(s1) * (1.0 / (_B * _N)) + jnp.sum(s2) * (1.0 / (_B * _M))
